# Initial kernel scaffold; baseline (speedup 1.0000x reference)
import jax
import jax.numpy as jnp
from jax.experimental import pallas as pl


def kernel(input, codebook):
    raise NotImplementedError("write your pallas kernel here")


# speedup vs baseline: 1.6476x; 1.6476x over previous
import functools

import jax
import jax.numpy as jnp
from jax import lax
from jax.experimental import pallas as pl
from jax.experimental.pallas import tpu as pltpu
from jax.experimental.pallas import tpu_sc as plsc

NUM_EMBEDDING = 1024
EMBEDDING_DIM = 64
COMMITMENT = 1.0

BT = 1024


def _tc_body(x_ref, cb_ref, dist_ref, enc_ref, loss_ref):
    pid = pl.program_id(0)
    nsteps = pl.num_programs(0)
    x = x_ref[...]
    c = cb_ref[...]
    xc = lax.dot_general(x, c, (((1,), (1,)), ((), ())),
                         preferred_element_type=jnp.float32)
    x2 = jnp.sum(x * x, axis=1, keepdims=True)
    c2 = jnp.sum(c * c, axis=1)[None, :]
    dist = x2 + c2 - 2.0 * xc
    dist_ref[...] = dist

    mind = jnp.min(dist, axis=1)
    k_iota = lax.broadcasted_iota(jnp.int32, dist.shape, 1)
    enc = jnp.min(jnp.where(dist == mind[:, None], k_iota, NUM_EMBEDDING),
                  axis=1)
    enc_ref[0, 0, :] = enc

    acc = loss_ref[...]
    prev = jnp.where(pid == 0, jnp.zeros_like(acc), acc)
    total = prev + jnp.sum(mind)
    scale = (1.0 + COMMITMENT) / (x_ref.shape[0] * nsteps * EMBEDDING_DIM)
    loss_ref[...] = jnp.where(pid == nsteps - 1, total * scale, total)


def _tc_distances(flat, codebook):
    n_tok = flat.shape[0]
    grid = n_tok // BT
    dist, enc, loss = pl.pallas_call(
        _tc_body,
        grid=(grid,),
        in_specs=[
            pl.BlockSpec((BT, EMBEDDING_DIM), lambda i: (i, 0)),
            pl.BlockSpec((NUM_EMBEDDING, EMBEDDING_DIM), lambda i: (0, 0)),
        ],
        out_specs=[
            pl.BlockSpec((BT, NUM_EMBEDDING), lambda i: (i, 0)),
            pl.BlockSpec((1, 1, BT), lambda i: (i, 0, 0)),
            pl.BlockSpec((1, 1), lambda i: (0, 0)),
        ],
        out_shape=[
            jax.ShapeDtypeStruct((n_tok, NUM_EMBEDDING), jnp.float32),
            jax.ShapeDtypeStruct((grid, 1, BT), jnp.int32),
            jax.ShapeDtypeStruct((1, 1), jnp.float32),
        ],
    )(flat, codebook)
    return dist, enc.reshape(n_tok), loss[0, 0]


def _sc_gather(codebook, enc):
    info = plsc.get_sparse_core_info()
    nc, ns = info.num_cores, info.num_subcores
    nw = nc * ns
    n_tok = enc.shape[0]
    b_per_w = n_tok // nw
    mesh = plsc.VectorSubcoreMesh(core_axis_name="c", subcore_axis_name="s")

    @functools.partial(
        pl.kernel,
        mesh=mesh,
        compiler_params=pltpu.CompilerParams(use_tc_tiling_on_sc=False),
        out_type=jax.ShapeDtypeStruct((n_tok, EMBEDDING_DIM), jnp.float32),
        scratch_types=[
            pltpu.VMEM((b_per_w,), jnp.int32),
            pltpu.VMEM((b_per_w, EMBEDDING_DIM), jnp.float32),
            pltpu.SemaphoreType.DMA,
        ],
    )
    def gather_k(table_hbm, idx_hbm, out_hbm, idx_v, rows_v, sem):
        wid = lax.axis_index("s") * nc + lax.axis_index("c")
        base = wid * b_per_w
        pltpu.sync_copy(idx_hbm.at[pl.ds(base, b_per_w)], idx_v)
        pltpu.async_copy(table_hbm.at[idx_v], rows_v, sem).wait()
        pltpu.sync_copy(rows_v, out_hbm.at[pl.ds(base, b_per_w)])

    return gather_k(codebook, enc)


def kernel(input, codebook):
    x = input
    flat = x.reshape(-1, EMBEDDING_DIM)
    distances, encoding, loss = _tc_distances(flat, codebook)
    quantized = _sc_gather(codebook, encoding).reshape(x.shape)
    quantized_st = x + lax.stop_gradient(quantized - x)
    return (quantized_st, encoding, distances, loss)


# speedup vs baseline: 1.7878x; 1.0851x over previous
import functools

import jax
import jax.numpy as jnp
from jax import lax
from jax.experimental import pallas as pl
from jax.experimental.pallas import tpu as pltpu
from jax.experimental.pallas import tpu_sc as plsc

NUM_EMBEDDING = 1024
EMBEDDING_DIM = 64
COMMITMENT = 1.0

BT = 2048


def _tc_body(x_ref, cb_ref, dist_ref, enc_ref, loss_ref):
    pid = pl.program_id(0)
    nsteps = pl.num_programs(0)
    x = x_ref[...]
    c = cb_ref[...]
    xc = lax.dot_general(x, c, (((1,), (1,)), ((), ())),
                         preferred_element_type=jnp.float32)
    x2 = jnp.sum(x * x, axis=1, keepdims=True)
    c2 = jnp.sum(c * c, axis=1)[None, :]
    dist = x2 + c2 - 2.0 * xc
    dist_ref[...] = dist

    mind = jnp.min(dist, axis=1)
    k_iota = lax.broadcasted_iota(jnp.int32, dist.shape, 1)
    enc = jnp.min(jnp.where(dist == mind[:, None], k_iota, NUM_EMBEDDING),
                  axis=1)
    enc_ref[0, 0, :] = enc

    acc = loss_ref[...]
    prev = jnp.where(pid == 0, jnp.zeros_like(acc), acc)
    total = prev + jnp.sum(mind)
    scale = (1.0 + COMMITMENT) / (x_ref.shape[0] * nsteps * EMBEDDING_DIM)
    loss_ref[...] = jnp.where(pid == nsteps - 1, total * scale, total)


def _tc_distances(flat, codebook):
    n_tok = flat.shape[0]
    grid = n_tok // BT
    dist, enc, loss = pl.pallas_call(
        _tc_body,
        grid=(grid,),
        in_specs=[
            pl.BlockSpec((BT, EMBEDDING_DIM), lambda i: (i, 0)),
            pl.BlockSpec((NUM_EMBEDDING, EMBEDDING_DIM), lambda i: (0, 0)),
        ],
        out_specs=[
            pl.BlockSpec((BT, NUM_EMBEDDING), lambda i: (i, 0)),
            pl.BlockSpec((1, 1, BT), lambda i: (i, 0, 0)),
            pl.BlockSpec((1, 1), lambda i: (0, 0)),
        ],
        out_shape=[
            jax.ShapeDtypeStruct((n_tok, NUM_EMBEDDING), jnp.float32),
            jax.ShapeDtypeStruct((grid, 1, BT), jnp.int32),
            jax.ShapeDtypeStruct((1, 1), jnp.float32),
        ],
    )(flat, codebook)
    return dist, enc.reshape(n_tok), loss[0, 0]


def _sc_gather(codebook, enc):
    info = plsc.get_sparse_core_info()
    nc, ns = info.num_cores, info.num_subcores
    nw = nc * ns
    n_tok = enc.shape[0]
    b_per_w = n_tok // nw
    mesh = plsc.VectorSubcoreMesh(core_axis_name="c", subcore_axis_name="s")

    @functools.partial(
        pl.kernel,
        mesh=mesh,
        compiler_params=pltpu.CompilerParams(use_tc_tiling_on_sc=False),
        out_type=jax.ShapeDtypeStruct((n_tok, EMBEDDING_DIM), jnp.float32),
        scratch_types=[
            pltpu.VMEM((b_per_w,), jnp.int32),
            pltpu.VMEM((b_per_w, EMBEDDING_DIM), jnp.float32),
            pltpu.SemaphoreType.DMA,
        ],
    )
    def gather_k(table_hbm, idx_hbm, out_hbm, idx_v, rows_v, sem):
        wid = lax.axis_index("s") * nc + lax.axis_index("c")
        base = wid * b_per_w
        pltpu.sync_copy(idx_hbm.at[pl.ds(base, b_per_w)], idx_v)
        pltpu.async_copy(table_hbm.at[idx_v], rows_v, sem).wait()
        pltpu.sync_copy(rows_v, out_hbm.at[pl.ds(base, b_per_w)])

    return gather_k(codebook, enc)


def kernel(input, codebook):
    x = input
    flat = x.reshape(-1, EMBEDDING_DIM)
    distances, encoding, loss = _tc_distances(flat, codebook)
    quantized = _sc_gather(codebook, encoding).reshape(x.shape)
    quantized_st = x + lax.stop_gradient(quantized - x)
    return (quantized_st, encoding, distances, loss)
